# baseline (speedup 1.0000x reference)
import jax
import jax.numpy as jnp
from jax import lax
from jax.experimental import pallas as pl
from jax.experimental.pallas import tpu as pltpu

N = 10000
D_IN = 128
D_OUT = 128
BB = 200
NB = 4
NOB = 2
NC = N // BB


def _adj_copy(adj_ref, buf_ref, in_sems, chunk, slot):
    return pltpu.make_async_copy(
        adj_ref.at[pl.ds(chunk * BB, BB), :], buf_ref.at[slot], in_sems.at[slot]
    )


def _out_copy(ostg_ref, out_ref, out_sems, chunk, oslot):
    return pltpu.make_async_copy(
        ostg_ref.at[oslot], out_ref.at[pl.ds(chunk * BB, BB), :], out_sems.at[oslot]
    )


def _gcn_kernel(x_ref, w_ref, adj_ref, out_ref,
                xv_ref, support_ref, buf_ref, ostg_ref,
                x_sem, in_sems, out_sems):
    x_copy = pltpu.make_async_copy(x_ref, xv_ref, x_sem)
    x_copy.start()
    for k in range(NB):
        _adj_copy(adj_ref, buf_ref, in_sems, k, k).start()
    x_copy.wait()
    support_ref[...] = jnp.dot(
        xv_ref[...], w_ref[...], preferred_element_type=jnp.float32
    )

    def body(c, _):
        slot = lax.rem(c, NB)
        oslot = lax.rem(c, NOB)
        _adj_copy(adj_ref, buf_ref, in_sems, c, slot).wait()
        result = jnp.maximum(
            jnp.dot(buf_ref[slot], support_ref[...],
                    preferred_element_type=jnp.float32),
            0.0,
        )

        @pl.when(c >= NOB)
        def _():
            _out_copy(ostg_ref, out_ref, out_sems, c - NOB, oslot).wait()

        ostg_ref[oslot] = result
        _out_copy(ostg_ref, out_ref, out_sems, c, oslot).start()

        @pl.when(c + NB < NC)
        def _():
            _adj_copy(adj_ref, buf_ref, in_sems, c + NB, slot).start()

        return _

    lax.fori_loop(0, NC, body, None)

    for k in range(NOB):
        chunk = NC - NOB + k
        _out_copy(ostg_ref, out_ref, out_sems, chunk, chunk % NOB).wait()


def kernel(input, adj, weight):
    return pl.pallas_call(
        _gcn_kernel,
        in_specs=[
            pl.BlockSpec(memory_space=pltpu.MemorySpace.HBM),
            pl.BlockSpec((D_IN, D_OUT), lambda: (0, 0)),
            pl.BlockSpec(memory_space=pltpu.MemorySpace.HBM),
        ],
        out_specs=pl.BlockSpec(memory_space=pltpu.MemorySpace.HBM),
        out_shape=jax.ShapeDtypeStruct((N, D_OUT), jnp.float32),
        scratch_shapes=[
            pltpu.VMEM((N, D_IN), jnp.float32),
            pltpu.VMEM((N, D_OUT), jnp.float32),
            pltpu.VMEM((NB, BB, N), jnp.float32),
            pltpu.VMEM((NOB, BB, D_OUT), jnp.float32),
            pltpu.SemaphoreType.DMA,
            pltpu.SemaphoreType.DMA((NB,)),
            pltpu.SemaphoreType.DMA((NOB,)),
        ],
    )(input, weight, adj)
